# baseline (speedup 1.0000x reference)
import jax
import jax.numpy as jnp
from jax.experimental import pallas as pl
from jax.experimental.pallas import tpu as pltpu

S, B, I = 512, 64, 1024
H = 512
NSLOT = 4
U = 4
NBLK = S // U

_INTERPRET = False


def _gru_step(h, xt, whru, whc):
    z = jnp.dot(h.astype(jnp.bfloat16), whru,
                preferred_element_type=jnp.float32)
    r = jax.nn.sigmoid(xt[0] + z[:, :H])
    u = jax.nn.sigmoid(xt[1] + z[:, H:])
    c = jnp.tanh(xt[2] + jnp.dot((r * h).astype(jnp.bfloat16), whc,
                                 preferred_element_type=jnp.float32))
    return u * h + (1.0 - u) * c


def _fused_kernel(xf_ref, xb_ref, wx_ref, b_ref, whru_ref, whc_ref, h0_ref,
                  out_ref, state_ref, h_scr, px, obuf, sems):
    i = pl.program_id(0)
    cur = jax.lax.rem(i, 2)
    prv = jax.lax.rem(i + 1, 2)
    slot = jax.lax.rem(i, NSLOT)

    @pl.when(i == 0)
    def _():
        h_scr[0] = h0_ref[:, :H]
        h_scr[1] = h0_ref[:, H:]

    xf2 = xf_ref[...].reshape(U * B, I).astype(jnp.bfloat16)
    xb2 = xb_ref[...].reshape(U * B, I).astype(jnp.bfloat16)
    for g in range(3):
        for d, x2 in ((0, xf2), (1, xb2)):
            res = jnp.dot(x2, wx_ref[d, g], preferred_element_type=jnp.float32)
            res = res + b_ref[0, (2 * g + d) * H:(2 * g + d + 1) * H]
            px[cur, d, g] = res.reshape(U, B, H)

    @pl.when(i > 0)
    def _():
        @pl.when(i > NSLOT)
        def _():
            for d in range(2):
                pltpu.make_async_copy(obuf.at[d, slot], obuf.at[d, slot],
                                      sems.at[d, slot]).wait()

        hf = h_scr[0]
        hb = h_scr[1]
        for k in range(U):
            hf = _gru_step(hf, [px[prv, 0, g, k] for g in range(3)],
                           whru_ref[0], whc_ref[0])
            hb = _gru_step(hb, [px[prv, 1, g, U - 1 - k] for g in range(3)],
                           whru_ref[1], whc_ref[1])
            obuf[0, slot, k] = hf
            obuf[1, slot, U - 1 - k] = hb
        h_scr[0] = hf
        h_scr[1] = hb

        pltpu.make_async_copy(
            obuf.at[0, slot],
            out_ref.at[pl.ds(U * (i - 1), U), :, pl.ds(0, H)],
            sems.at[0, slot]).start()
        pltpu.make_async_copy(
            obuf.at[1, slot],
            out_ref.at[pl.ds(S - U - U * (i - 1), U), :, pl.ds(H, H)],
            sems.at[1, slot]).start()

        @pl.when(i == NBLK)
        def _():
            state_ref[:, :H] = h_scr[0]
            state_ref[:, H:] = h_scr[1]
            for d in range(2):
                for s_ in range(NSLOT):
                    pltpu.make_async_copy(obuf.at[d, s_], obuf.at[d, s_],
                                          sems.at[d, s_]).wait()


def kernel(x, initial_state, Wx_f, Wh_f, b_f, Wx_b, Wh_b, b_b):
    Wx = jnp.stack([Wx_f, Wx_b]).astype(jnp.bfloat16)
    bias = jnp.stack([b_f, b_b], axis=1).reshape(1, 6 * H)
    Wh_ru = jnp.stack([
        jnp.concatenate([Wh_f[0], Wh_f[1]], axis=-1),
        jnp.concatenate([Wh_b[0], Wh_b[1]], axis=-1),
    ]).astype(jnp.bfloat16)
    Wh_c = jnp.stack([Wh_f[2], Wh_b[2]]).astype(jnp.bfloat16)

    last = NBLK - 1
    out, state = pl.pallas_call(
        _fused_kernel,
        grid=(NBLK + 1,),
        in_specs=[
            pl.BlockSpec((U, B, I), lambda i: (jnp.minimum(i, last), 0, 0)),
            pl.BlockSpec((U, B, I),
                         lambda i: (last - jnp.minimum(i, last), 0, 0)),
            pl.BlockSpec((2, 3, I, H), lambda i: (0, 0, 0, 0)),
            pl.BlockSpec((1, 6 * H), lambda i: (0, 0)),
            pl.BlockSpec((2, H, 2 * H), lambda i: (0, 0, 0)),
            pl.BlockSpec((2, H, H), lambda i: (0, 0, 0)),
            pl.BlockSpec((B, 2 * H), lambda i: (0, 0)),
        ],
        out_specs=[
            pl.BlockSpec(memory_space=pl.ANY),
            pl.BlockSpec((B, 2 * H), lambda i: (0, 0)),
        ],
        out_shape=[
            jax.ShapeDtypeStruct((S, B, 2 * H), jnp.float32),
            jax.ShapeDtypeStruct((B, 2 * H), jnp.float32),
        ],
        scratch_shapes=[
            pltpu.VMEM((2, B, H), jnp.float32),
            pltpu.VMEM((2, 2, 3, U, B, H), jnp.float32),
            pltpu.VMEM((2, NSLOT, U, B, H), jnp.float32),
            pltpu.SemaphoreType.DMA((2, NSLOT)),
        ],
        compiler_params=pltpu.CompilerParams(
            dimension_semantics=("arbitrary",),
            vmem_limit_bytes=56 * 1024 * 1024,
        ),
        name="bigru_fused",
        interpret=_INTERPRET,
    )(x, x, Wx, bias, Wh_ru, Wh_c, initial_state)

    return out, state


# speedup vs baseline: 1.0058x; 1.0058x over previous
import jax
import jax.numpy as jnp
from jax.experimental import pallas as pl
from jax.experimental.pallas import tpu as pltpu

S, B, I = 512, 64, 1024
H = 512
NSLOT = 4
U = 4
NBLK = S // U

_INTERPRET = False


def _gru_step(h, xt, whru, whc):
    z = jnp.dot(h.astype(jnp.bfloat16), whru,
                preferred_element_type=jnp.float32)
    r = jax.nn.sigmoid(xt[0] + z[:, :H])
    u = jax.nn.sigmoid(xt[1] + z[:, H:])
    c = jnp.tanh(xt[2] + jnp.dot((r * h).astype(jnp.bfloat16), whc,
                                 preferred_element_type=jnp.float32))
    return u * h + (1.0 - u) * c


def _fused_kernel(xf_ref, xb_ref, wx_ref, b_ref, whru_ref, whc_ref, h0_ref,
                  out_ref, state_ref, h_scr, px, obuf, sems):
    i = pl.program_id(0)
    cur = jax.lax.rem(i, 2)
    prv = jax.lax.rem(i + 1, 2)
    slot = jax.lax.rem(i, NSLOT)

    @pl.when(i == 0)
    def _():
        h_scr[0] = h0_ref[:, :H]
        h_scr[1] = h0_ref[:, H:]

    xf2 = xf_ref[...].reshape(U * B, I).astype(jnp.bfloat16)
    xb2 = xb_ref[...].reshape(U * B, I).astype(jnp.bfloat16)
    for g in range(3):
        for d, x2 in ((0, xf2), (1, xb2)):
            res = jnp.dot(x2, wx_ref[d, g], preferred_element_type=jnp.float32)
            res = res + b_ref[0, (2 * g + d) * H:(2 * g + d + 1) * H]
            px[cur, d, g] = res.reshape(U, B, H)


    @pl.when(i > NSLOT)
    def _():
        for d in range(2):
            pltpu.make_async_copy(obuf.at[d, slot], obuf.at[d, slot],
                                  sems.at[d, slot]).wait()

    hf = h_scr[0]
    hb = h_scr[1]
    for k in range(U):
        hf = _gru_step(hf, [px[prv, 0, g, k] for g in range(3)],
                       whru_ref[0], whc_ref[0])
        hb = _gru_step(hb, [px[prv, 1, g, U - 1 - k] for g in range(3)],
                       whru_ref[1], whc_ref[1])
        obuf[0, slot, k] = hf
        obuf[1, slot, U - 1 - k] = hb

    @pl.when(i > 0)
    def _():
        h_scr[0] = hf
        h_scr[1] = hb
        pltpu.make_async_copy(
            obuf.at[0, slot],
            out_ref.at[pl.ds(U * (i - 1), U), :, pl.ds(0, H)],
            sems.at[0, slot]).start()
        pltpu.make_async_copy(
            obuf.at[1, slot],
            out_ref.at[pl.ds(S - U - U * (i - 1), U), :, pl.ds(H, H)],
            sems.at[1, slot]).start()

    @pl.when(i == NBLK)
    def _():
        state_ref[:, :H] = h_scr[0]
        state_ref[:, H:] = h_scr[1]
        for d in range(2):
            for s_ in range(NSLOT):
                pltpu.make_async_copy(obuf.at[d, s_], obuf.at[d, s_],
                                      sems.at[d, s_]).wait()


def kernel(x, initial_state, Wx_f, Wh_f, b_f, Wx_b, Wh_b, b_b):
    Wx = jnp.stack([Wx_f, Wx_b]).astype(jnp.bfloat16)
    bias = jnp.stack([b_f, b_b], axis=1).reshape(1, 6 * H)
    Wh_ru = jnp.stack([
        jnp.concatenate([Wh_f[0], Wh_f[1]], axis=-1),
        jnp.concatenate([Wh_b[0], Wh_b[1]], axis=-1),
    ]).astype(jnp.bfloat16)
    Wh_c = jnp.stack([Wh_f[2], Wh_b[2]]).astype(jnp.bfloat16)

    last = NBLK - 1
    out, state = pl.pallas_call(
        _fused_kernel,
        grid=(NBLK + 1,),
        in_specs=[
            pl.BlockSpec((U, B, I), lambda i: (jnp.minimum(i, last), 0, 0)),
            pl.BlockSpec((U, B, I),
                         lambda i: (last - jnp.minimum(i, last), 0, 0)),
            pl.BlockSpec((2, 3, I, H), lambda i: (0, 0, 0, 0)),
            pl.BlockSpec((1, 6 * H), lambda i: (0, 0)),
            pl.BlockSpec((2, H, 2 * H), lambda i: (0, 0, 0)),
            pl.BlockSpec((2, H, H), lambda i: (0, 0, 0)),
            pl.BlockSpec((B, 2 * H), lambda i: (0, 0)),
        ],
        out_specs=[
            pl.BlockSpec(memory_space=pl.ANY),
            pl.BlockSpec((B, 2 * H), lambda i: (0, 0)),
        ],
        out_shape=[
            jax.ShapeDtypeStruct((S, B, 2 * H), jnp.float32),
            jax.ShapeDtypeStruct((B, 2 * H), jnp.float32),
        ],
        scratch_shapes=[
            pltpu.VMEM((2, B, H), jnp.float32),
            pltpu.VMEM((2, 2, 3, U, B, H), jnp.float32),
            pltpu.VMEM((2, NSLOT, U, B, H), jnp.float32),
            pltpu.SemaphoreType.DMA((2, NSLOT)),
        ],
        compiler_params=pltpu.CompilerParams(
            dimension_semantics=("arbitrary",),
            vmem_limit_bytes=56 * 1024 * 1024,
        ),
        name="bigru_fused",
        interpret=_INTERPRET,
    )(x, x, Wx, bias, Wh_ru, Wh_c, initial_state)

    return out, state


# speedup vs baseline: 1.0766x; 1.0705x over previous
import jax
import jax.numpy as jnp
from jax.experimental import pallas as pl
from jax.experimental.pallas import tpu as pltpu

S, B, I = 512, 64, 1024
H = 512
TS = 16
NSLOT = 8
U = 8

_INTERPRET = False


def _proj_kernel(x_ref, w_ref, b_ref, out_ref):
    x2 = x_ref[...].reshape(TS * B, I).astype(jnp.bfloat16)
    for g in range(3):
        for d in range(2):
            res = jnp.dot(x2, w_ref[d, g], preferred_element_type=jnp.float32)
            res = res + b_ref[0, (2 * g + d) * H:(2 * g + d + 1) * H]
            out_ref[g, :, :, d * H:(d + 1) * H] = (
                res.astype(jnp.bfloat16).reshape(TS, B, H))


def _gru_step(h, xt, whru, whc):
    z = jnp.dot(h.astype(jnp.bfloat16), whru,
                preferred_element_type=jnp.float32)
    r = jax.nn.sigmoid(xt[0] + z[:, :H])
    u = jax.nn.sigmoid(xt[1] + z[:, H:])
    c = jnp.tanh(xt[2] + jnp.dot((r * h).astype(jnp.bfloat16), whc,
                                 preferred_element_type=jnp.float32))
    return u * h + (1.0 - u) * c


def _rec_kernel(xgf_ref, xgb_ref, whru_ref, whc_ref, h0_ref,
                out_ref, state_ref, h_scr, obuf, sems):
    i = pl.program_id(0)
    slot = jax.lax.rem(i, NSLOT)

    @pl.when(i == 0)
    def _():
        h_scr[0] = h0_ref[:, :H]
        h_scr[1] = h0_ref[:, H:]

    @pl.when(i >= NSLOT)
    def _():
        for d in range(2):
            pltpu.make_async_copy(obuf.at[d, slot], obuf.at[d, slot],
                                  sems.at[d, slot]).wait()

    hf = h_scr[0]
    hb = h_scr[1]
    for k in range(U):
        hf = _gru_step(hf, [xgf_ref[g, k] for g in range(3)],
                       whru_ref[0], whc_ref[0])
        hb = _gru_step(hb, [xgb_ref[g, U - 1 - k] for g in range(3)],
                       whru_ref[1], whc_ref[1])
        obuf[0, slot, k] = hf
        obuf[1, slot, U - 1 - k] = hb
    h_scr[0] = hf
    h_scr[1] = hb

    pltpu.make_async_copy(obuf.at[0, slot],
                          out_ref.at[pl.ds(U * i, U), :, pl.ds(0, H)],
                          sems.at[0, slot]).start()
    pltpu.make_async_copy(obuf.at[1, slot],
                          out_ref.at[pl.ds(S - U - U * i, U), :, pl.ds(H, H)],
                          sems.at[1, slot]).start()

    @pl.when(i == S // U - 1)
    def _():
        state_ref[:, :H] = hf
        state_ref[:, H:] = hb
        for d in range(2):
            for s_ in range(NSLOT):
                pltpu.make_async_copy(obuf.at[d, s_], obuf.at[d, s_],
                                      sems.at[d, s_]).wait()


def kernel(x, initial_state, Wx_f, Wh_f, b_f, Wx_b, Wh_b, b_b):
    Wx = jnp.stack([Wx_f, Wx_b]).astype(jnp.bfloat16)
    bias = jnp.stack([b_f, b_b], axis=1).reshape(1, 6 * H)
    Wh_ru = jnp.stack([
        jnp.concatenate([Wh_f[0], Wh_f[1]], axis=-1),
        jnp.concatenate([Wh_b[0], Wh_b[1]], axis=-1),
    ]).astype(jnp.bfloat16)
    Wh_c = jnp.stack([Wh_f[2], Wh_b[2]]).astype(jnp.bfloat16)

    xg = pl.pallas_call(
        _proj_kernel,
        grid=(S // TS,),
        in_specs=[
            pl.BlockSpec((TS, B, I), lambda si: (si, 0, 0)),
            pl.BlockSpec((2, 3, I, H), lambda si: (0, 0, 0, 0)),
            pl.BlockSpec((1, 6 * H), lambda si: (0, 0)),
        ],
        out_specs=pl.BlockSpec((3, TS, B, 2 * H), lambda si: (0, si, 0, 0)),
        out_shape=jax.ShapeDtypeStruct((3, S, B, 2 * H), jnp.bfloat16),
        compiler_params=pltpu.CompilerParams(
            dimension_semantics=("arbitrary",),
            vmem_limit_bytes=56 * 1024 * 1024,
        ),
        name="bigru_proj",
        interpret=_INTERPRET,
    )(x, Wx, bias)

    out, state = pl.pallas_call(
        _rec_kernel,
        grid=(S // U,),
        in_specs=[
            pl.BlockSpec((3, U, B, H), lambda i: (0, i, 0, 0)),
            pl.BlockSpec((3, U, B, H), lambda i: (0, S // U - 1 - i, 0, 1)),
            pl.BlockSpec((2, H, 2 * H), lambda i: (0, 0, 0)),
            pl.BlockSpec((2, H, H), lambda i: (0, 0, 0)),
            pl.BlockSpec((B, 2 * H), lambda i: (0, 0)),
        ],
        out_specs=[
            pl.BlockSpec(memory_space=pl.ANY),
            pl.BlockSpec((B, 2 * H), lambda i: (0, 0)),
        ],
        out_shape=[
            jax.ShapeDtypeStruct((S, B, 2 * H), jnp.float32),
            jax.ShapeDtypeStruct((B, 2 * H), jnp.float32),
        ],
        scratch_shapes=[
            pltpu.VMEM((2, B, H), jnp.float32),
            pltpu.VMEM((2, NSLOT, U, B, H), jnp.float32),
            pltpu.SemaphoreType.DMA((2, NSLOT)),
        ],
        compiler_params=pltpu.CompilerParams(
            dimension_semantics=("arbitrary",),
            vmem_limit_bytes=56 * 1024 * 1024,
        ),
        name="bigru_rec",
        interpret=_INTERPRET,
    )(xg, xg, Wh_ru, Wh_c, initial_state)

    return out, state


# speedup vs baseline: 1.0821x; 1.0051x over previous
import jax
import jax.numpy as jnp
from jax.experimental import pallas as pl
from jax.experimental.pallas import tpu as pltpu

S, B, I = 512, 64, 1024
H = 512
TS = 16
NSLOT = 4
U = 16

_INTERPRET = False


def _proj_kernel(x_ref, w_ref, b_ref, out_ref):
    x2 = x_ref[...].reshape(TS * B, I).astype(jnp.bfloat16)
    for g in range(3):
        for d in range(2):
            res = jnp.dot(x2, w_ref[d, g], preferred_element_type=jnp.float32)
            res = res + b_ref[0, (2 * g + d) * H:(2 * g + d + 1) * H]
            out_ref[g, :, :, d * H:(d + 1) * H] = (
                res.astype(jnp.bfloat16).reshape(TS, B, H))


def _gru_step(h, xt, whru, whc):
    z = jnp.dot(h.astype(jnp.bfloat16), whru,
                preferred_element_type=jnp.float32)
    r = jax.nn.sigmoid(xt[0] + z[:, :H])
    u = jax.nn.sigmoid(xt[1] + z[:, H:])
    c = jnp.tanh(xt[2] + jnp.dot((r * h).astype(jnp.bfloat16), whc,
                                 preferred_element_type=jnp.float32))
    return u * h + (1.0 - u) * c


def _rec_kernel(xgf_ref, xgb_ref, whru_ref, whc_ref, h0_ref,
                out_ref, state_ref, h_scr, obuf, sems):
    i = pl.program_id(0)
    slot = jax.lax.rem(i, NSLOT)

    @pl.when(i == 0)
    def _():
        h_scr[0] = h0_ref[:, :H]
        h_scr[1] = h0_ref[:, H:]

    @pl.when(i >= NSLOT)
    def _():
        for d in range(2):
            pltpu.make_async_copy(obuf.at[d, slot], obuf.at[d, slot],
                                  sems.at[d, slot]).wait()

    hf = h_scr[0]
    hb = h_scr[1]
    for k in range(U):
        hf = _gru_step(hf, [xgf_ref[g, k] for g in range(3)],
                       whru_ref[0], whc_ref[0])
        hb = _gru_step(hb, [xgb_ref[g, U - 1 - k] for g in range(3)],
                       whru_ref[1], whc_ref[1])
        obuf[0, slot, k] = hf
        obuf[1, slot, U - 1 - k] = hb
    h_scr[0] = hf
    h_scr[1] = hb

    pltpu.make_async_copy(obuf.at[0, slot],
                          out_ref.at[pl.ds(U * i, U), :, pl.ds(0, H)],
                          sems.at[0, slot]).start()
    pltpu.make_async_copy(obuf.at[1, slot],
                          out_ref.at[pl.ds(S - U - U * i, U), :, pl.ds(H, H)],
                          sems.at[1, slot]).start()

    @pl.when(i == S // U - 1)
    def _():
        state_ref[:, :H] = hf
        state_ref[:, H:] = hb
        for d in range(2):
            for s_ in range(NSLOT):
                pltpu.make_async_copy(obuf.at[d, s_], obuf.at[d, s_],
                                      sems.at[d, s_]).wait()


def kernel(x, initial_state, Wx_f, Wh_f, b_f, Wx_b, Wh_b, b_b):
    Wx = jnp.stack([Wx_f, Wx_b]).astype(jnp.bfloat16)
    bias = jnp.stack([b_f, b_b], axis=1).reshape(1, 6 * H)
    Wh_ru = jnp.stack([
        jnp.concatenate([Wh_f[0], Wh_f[1]], axis=-1),
        jnp.concatenate([Wh_b[0], Wh_b[1]], axis=-1),
    ]).astype(jnp.bfloat16)
    Wh_c = jnp.stack([Wh_f[2], Wh_b[2]]).astype(jnp.bfloat16)

    xg = pl.pallas_call(
        _proj_kernel,
        grid=(S // TS,),
        in_specs=[
            pl.BlockSpec((TS, B, I), lambda si: (si, 0, 0)),
            pl.BlockSpec((2, 3, I, H), lambda si: (0, 0, 0, 0)),
            pl.BlockSpec((1, 6 * H), lambda si: (0, 0)),
        ],
        out_specs=pl.BlockSpec((3, TS, B, 2 * H), lambda si: (0, si, 0, 0)),
        out_shape=jax.ShapeDtypeStruct((3, S, B, 2 * H), jnp.bfloat16),
        compiler_params=pltpu.CompilerParams(
            dimension_semantics=("arbitrary",),
            vmem_limit_bytes=56 * 1024 * 1024,
        ),
        name="bigru_proj",
        interpret=_INTERPRET,
    )(x, Wx, bias)

    out, state = pl.pallas_call(
        _rec_kernel,
        grid=(S // U,),
        in_specs=[
            pl.BlockSpec((3, U, B, H), lambda i: (0, i, 0, 0)),
            pl.BlockSpec((3, U, B, H), lambda i: (0, S // U - 1 - i, 0, 1)),
            pl.BlockSpec((2, H, 2 * H), lambda i: (0, 0, 0)),
            pl.BlockSpec((2, H, H), lambda i: (0, 0, 0)),
            pl.BlockSpec((B, 2 * H), lambda i: (0, 0)),
        ],
        out_specs=[
            pl.BlockSpec(memory_space=pl.ANY),
            pl.BlockSpec((B, 2 * H), lambda i: (0, 0)),
        ],
        out_shape=[
            jax.ShapeDtypeStruct((S, B, 2 * H), jnp.float32),
            jax.ShapeDtypeStruct((B, 2 * H), jnp.float32),
        ],
        scratch_shapes=[
            pltpu.VMEM((2, B, H), jnp.float32),
            pltpu.VMEM((2, NSLOT, U, B, H), jnp.float32),
            pltpu.SemaphoreType.DMA((2, NSLOT)),
        ],
        compiler_params=pltpu.CompilerParams(
            dimension_semantics=("arbitrary",),
            vmem_limit_bytes=56 * 1024 * 1024,
        ),
        name="bigru_rec",
        interpret=_INTERPRET,
    )(xg, xg, Wh_ru, Wh_c, initial_state)

    return out, state
